# baseline (speedup 1.0000x reference)
import jax
import jax.numpy as jnp
from jax import lax
from jax.experimental import pallas as pl
from jax.experimental.pallas import tpu as pltpu
from jax.experimental.pallas import tpu_sc as plsc

CLS_ID = 1
B = 16
L = 4096
LP1 = L + 1
NLANE = 16
OUT_PAD = 33 * 128
OUT0 = 2048
OUT1 = OUT_PAD - OUT0
IN1_OFF = OUT0 - 128
IN_V = 2304


def _body(values_hbm, lengths_hbm, out_hbm, nl_hbm, in_v, out_v, len_v, nl_v):
    c = lax.axis_index("c")
    s = lax.axis_index("s")
    wid = s * 2 + c
    row = wid % B
    half = wid // B

    pltpu.sync_copy(lengths_hbm, len_v)
    lane = lax.iota(jnp.int32, NLANE)
    len_vec = len_v[...]
    my_len = jnp.sum(jnp.where(lane == row, len_vec, 0))

    @pl.when(half == 0)
    def _h0():
        pltpu.sync_copy(values_hbm.at[row, pl.ds(0, OUT0)], in_v.at[pl.ds(0, OUT0)])

        @plsc.parallel_loop(0, OUT0, step=NLANE, unroll=8)
        def _shift(j):
            v = in_v[pl.ds(j, NLANE)]
            out_v[pl.ds(j + 1, NLANE)] = jnp.where(lane + j < my_len, v, 0)

        head = out_v[pl.ds(0, NLANE)]
        out_v[pl.ds(0, NLANE)] = jnp.where(lane == 0, CLS_ID, head)
        pltpu.sync_copy(out_v.at[pl.ds(0, OUT0)], out_hbm.at[row, pl.ds(0, OUT0)])

        @pl.when(row == 0)
        def _newlen():
            nl_v[...] = len_vec + 1
            pltpu.sync_copy(nl_v, nl_hbm)

    @pl.when(half == 1)
    def _h1():
        pltpu.sync_copy(
            values_hbm.at[row, pl.ds(IN1_OFF, OUT1)], in_v.at[pl.ds(0, OUT1)]
        )

        @plsc.parallel_loop(0, OUT1, step=NLANE, unroll=8)
        def _shift(j):
            v = in_v[pl.ds(j + 127, NLANE)]
            p = lane + j + (OUT0 - 1)
            out_v[pl.ds(j, NLANE)] = jnp.where(p < my_len, v, 0)

        pltpu.sync_copy(
            out_v.at[pl.ds(0, OUT1)], out_hbm.at[row, pl.ds(OUT0, OUT1)]
        )


_mesh = plsc.VectorSubcoreMesh(core_axis_name="c", subcore_axis_name="s")

_prepend = pl.kernel(
    _body,
    out_type=[
        jax.ShapeDtypeStruct((B, OUT_PAD), jnp.int32),
        jax.ShapeDtypeStruct((B,), jnp.int32),
    ],
    mesh=_mesh,
    compiler_params=pltpu.CompilerParams(
        needs_layout_passes=False, skip_device_barrier=True
    ),
    scratch_types=[
        pltpu.VMEM((IN_V,), jnp.int32),
        pltpu.VMEM((OUT1 + NLANE,), jnp.int32),
        pltpu.VMEM((NLANE,), jnp.int32),
        pltpu.VMEM((NLANE,), jnp.int32),
    ],
)


def kernel(values, lengths):
    out_pad, new_lengths = _prepend(
        values.astype(jnp.int32), lengths.astype(jnp.int32)
    )
    out = out_pad[:, :LP1].astype(values.dtype)
    return out, new_lengths.astype(lengths.dtype)


# speedup vs baseline: 1.0420x; 1.0420x over previous
import jax
import jax.numpy as jnp
from jax import lax
from jax.experimental import pallas as pl
from jax.experimental.pallas import tpu as pltpu
from jax.experimental.pallas import tpu_sc as plsc

CLS_ID = 1
B = 16
L = 4096
LP1 = L + 1
NLANE = 16
BLK = 128
NBLK = L // BLK
VPB = BLK // NLANE
OUT_PAD = 33 * 128


def _body(values_hbm, lengths_hbm, out_hbm, nl_hbm, in_v, out_v, len_v, nl_v):
    row = lax.axis_index("s")
    pltpu.sync_copy(values_hbm.at[row], in_v)
    pltpu.sync_copy(lengths_hbm, len_v)
    lane = lax.iota(jnp.int32, NLANE)
    len_vec = len_v[...]
    my_len = jnp.sum(jnp.where(lane == row, len_vec, 0))
    nb = my_len // BLK

    def _copy_blk(k, carry):
        base = k * BLK
        for i in range(VPB):
            j = base + i * NLANE
            out_v[pl.ds(j + 1, NLANE)] = in_v[pl.ds(j, NLANE)]
        return carry

    lax.fori_loop(0, nb, _copy_blk, 0)

    bbase = nb * BLK
    for i in range(VPB):
        j = bbase + i * NLANE
        v = in_v[pl.ds(j, NLANE)]
        out_v[pl.ds(j + 1, NLANE)] = jnp.where(lane + j < my_len, v, 0)

    zero = jnp.zeros((NLANE,), jnp.int32)

    def _zero_blk(k, carry):
        base = k * BLK
        for i in range(VPB):
            out_v[pl.ds(base + i * NLANE + 1, NLANE)] = zero
        return carry

    lax.fori_loop(nb + 1, NBLK, _zero_blk, 0)

    head = out_v[pl.ds(0, NLANE)]
    out_v[pl.ds(0, NLANE)] = jnp.where(lane == 0, CLS_ID, head)
    pltpu.sync_copy(out_v, out_hbm.at[row])

    @pl.when(row == 0)
    def _newlen():
        nl_v[...] = len_vec + 1
        pltpu.sync_copy(nl_v, nl_hbm)


_mesh = plsc.VectorSubcoreMesh(
    core_axis_name="c", subcore_axis_name="s", num_cores=1
)

_prepend = pl.kernel(
    _body,
    out_type=[
        jax.ShapeDtypeStruct((B, OUT_PAD), jnp.int32),
        jax.ShapeDtypeStruct((B,), jnp.int32),
    ],
    mesh=_mesh,
    compiler_params=pltpu.CompilerParams(
        needs_layout_passes=False, skip_device_barrier=True
    ),
    scratch_types=[
        pltpu.VMEM((L,), jnp.int32),
        pltpu.VMEM((OUT_PAD,), jnp.int32),
        pltpu.VMEM((NLANE,), jnp.int32),
        pltpu.VMEM((NLANE,), jnp.int32),
    ],
)


def kernel(values, lengths):
    out_pad, new_lengths = _prepend(
        values.astype(jnp.int32), lengths.astype(jnp.int32)
    )
    out = out_pad[:, :LP1].astype(values.dtype)
    return out, new_lengths.astype(lengths.dtype)


# speedup vs baseline: 1.0726x; 1.0293x over previous
import jax
import jax.numpy as jnp
from jax import lax
from jax.experimental import pallas as pl
from jax.experimental.pallas import tpu as pltpu
from jax.experimental.pallas import tpu_sc as plsc

CLS_ID = 1
B = 16
L = 4096
LP1 = L + 1
NLANE = 16
BLK = 128
NBLK = L // BLK
VPB = BLK // NLANE
OUT_PAD = 33 * 128


def _body(values_hbm, lengths_hbm, out_hbm, nl_hbm, in_v, out_v, len_v, nl_v):
    row = lax.axis_index("s")
    pltpu.sync_copy(values_hbm.at[row], in_v)
    pltpu.sync_copy(lengths_hbm, len_v)
    lane = lax.iota(jnp.int32, NLANE)
    len_vec = len_v[...]
    my_len = jnp.sum(jnp.where(lane == row, len_vec, 0))
    nb = my_len // BLK

    @plsc.parallel_loop(0, nb * BLK, step=BLK)
    def _copy_blk(base):
        for i in range(VPB):
            j = base + i * NLANE
            out_v[pl.ds(j + 1, NLANE)] = in_v[pl.ds(j, NLANE)]

    bbase = nb * BLK
    for i in range(VPB):
        j = bbase + i * NLANE
        v = in_v[pl.ds(j, NLANE)]
        out_v[pl.ds(j + 1, NLANE)] = jnp.where(lane + j < my_len, v, 0)

    zero = jnp.zeros((NLANE,), jnp.int32)

    @plsc.parallel_loop((nb + 1) * BLK, L, step=BLK)
    def _zero_blk(base):
        for i in range(VPB):
            out_v[pl.ds(base + i * NLANE + 1, NLANE)] = zero

    head = out_v[pl.ds(0, NLANE)]
    out_v[pl.ds(0, NLANE)] = jnp.where(lane == 0, CLS_ID, head)
    pltpu.sync_copy(out_v, out_hbm.at[row])

    @pl.when(row == 0)
    def _newlen():
        nl_v[...] = len_vec + 1
        pltpu.sync_copy(nl_v, nl_hbm)


_mesh = plsc.VectorSubcoreMesh(
    core_axis_name="c", subcore_axis_name="s", num_cores=1
)

_prepend = pl.kernel(
    _body,
    out_type=[
        jax.ShapeDtypeStruct((B, OUT_PAD), jnp.int32),
        jax.ShapeDtypeStruct((B,), jnp.int32),
    ],
    mesh=_mesh,
    compiler_params=pltpu.CompilerParams(
        needs_layout_passes=False, skip_device_barrier=True
    ),
    scratch_types=[
        pltpu.VMEM((L,), jnp.int32),
        pltpu.VMEM((OUT_PAD,), jnp.int32),
        pltpu.VMEM((NLANE,), jnp.int32),
        pltpu.VMEM((NLANE,), jnp.int32),
    ],
)


def kernel(values, lengths):
    out_pad, new_lengths = _prepend(
        values.astype(jnp.int32), lengths.astype(jnp.int32)
    )
    out = out_pad[:, :LP1].astype(values.dtype)
    return out, new_lengths.astype(lengths.dtype)


# speedup vs baseline: 1.0830x; 1.0097x over previous
import jax
import jax.numpy as jnp
from jax import lax
from jax.experimental import pallas as pl
from jax.experimental.pallas import tpu as pltpu
from jax.experimental.pallas import tpu_sc as plsc

CLS_ID = 1
B = 16
L = 4096
LP1 = L + 1
NLANE = 16
OUT_PAD = 33 * 128


def _body(values_hbm, lengths_hbm, out_hbm, nl_hbm, in_v, out_v, len_v, nl_v):
    row = lax.axis_index("s")
    pltpu.sync_copy(values_hbm.at[row], in_v)
    pltpu.sync_copy(lengths_hbm, len_v)
    lane = lax.iota(jnp.int32, NLANE)
    len_vec = len_v[...]
    my_len = jnp.sum(jnp.where(lane == row, len_vec, 0))

    @plsc.parallel_loop(0, L, step=NLANE, unroll=8)
    def _shift(j):
        v = in_v[pl.ds(j, NLANE)]
        out_v[pl.ds(j + 1, NLANE)] = jnp.where(lane + j < my_len, v, 0)

    head = out_v[pl.ds(0, NLANE)]
    out_v[pl.ds(0, NLANE)] = jnp.where(lane == 0, CLS_ID, head)
    pltpu.sync_copy(out_v, out_hbm.at[row])

    @pl.when(row == 0)
    def _newlen():
        nl_v[...] = len_vec + 1
        pltpu.sync_copy(nl_v, nl_hbm)


_mesh = plsc.VectorSubcoreMesh(
    core_axis_name="c", subcore_axis_name="s", num_cores=1
)

_prepend = pl.kernel(
    _body,
    out_type=[
        jax.ShapeDtypeStruct((B, OUT_PAD), jnp.int32),
        jax.ShapeDtypeStruct((B,), jnp.int32),
    ],
    mesh=_mesh,
    compiler_params=pltpu.CompilerParams(
        needs_layout_passes=False, skip_device_barrier=True
    ),
    scratch_types=[
        pltpu.VMEM((L,), jnp.int32),
        pltpu.VMEM((OUT_PAD,), jnp.int32),
        pltpu.VMEM((NLANE,), jnp.int32),
        pltpu.VMEM((NLANE,), jnp.int32),
    ],
)


def kernel(values, lengths):
    out_pad, new_lengths = _prepend(
        values.astype(jnp.int32), lengths.astype(jnp.int32)
    )
    out = out_pad[:, :LP1].astype(values.dtype)
    return out, new_lengths.astype(lengths.dtype)
